# Initial kernel scaffold; baseline (speedup 1.0000x reference)
import jax
import jax.numpy as jnp
from jax.experimental import pallas as pl


def kernel(x, Wg, Wn, noise):
    raise NotImplementedError("write your pallas kernel here")


# speedup vs baseline: 4.2108x; 4.2108x over previous
import functools

import jax
import jax.numpy as jnp
from jax.experimental import pallas as pl
from jax.experimental.pallas import tpu as pltpu

NUM_EXPERTS = 64
TOP_K = 8
TM = 512


def _router_body(x_ref, wgt_ref, wnt_ref, noise_ref, gates_ref, experts_ref):
    x = x_ref[...]
    g = jnp.dot(x, wgt_ref[...], preferred_element_type=jnp.float32)
    npre = jnp.dot(x, wnt_ref[...], preferred_element_type=jnp.float32)
    logits = g + noise_ref[...] * jax.nn.softplus(npre)

    col = jax.lax.broadcasted_iota(jnp.int32, logits.shape, 1)
    neg_inf = jnp.float32(-jnp.inf)
    work = logits
    vals = []
    idxs = []
    for _ in range(TOP_K):
        m = jnp.max(work, axis=1, keepdims=True)
        idx = jnp.min(jnp.where(work == m, col, NUM_EXPERTS), axis=1,
                      keepdims=True)
        vals.append(m)
        idxs.append(idx)
        work = jnp.where(col == idx, neg_inf, work)
    top_idx = jnp.concatenate(idxs, axis=1)

    sel = work == neg_inf
    e = jnp.where(sel, jnp.exp(logits - vals[0]), 0.0)
    gates_ref[...] = e / jnp.sum(e, axis=1, keepdims=True)
    experts_ref[...] = top_idx


@jax.jit
def kernel(x, Wg, Wn, noise):
    n_tokens = x.shape[0]
    grid = (n_tokens // TM,)
    gates, experts = pl.pallas_call(
        _router_body,
        grid=grid,
        in_specs=[
            pl.BlockSpec((TM, x.shape[1]), lambda i: (i, 0)),
            pl.BlockSpec((x.shape[1], NUM_EXPERTS), lambda i: (0, 0)),
            pl.BlockSpec((x.shape[1], NUM_EXPERTS), lambda i: (0, 0)),
            pl.BlockSpec((TM, NUM_EXPERTS), lambda i: (i, 0)),
        ],
        out_specs=[
            pl.BlockSpec((TM, NUM_EXPERTS), lambda i: (i, 0)),
            pl.BlockSpec((TM, TOP_K), lambda i: (i, 0)),
        ],
        out_shape=[
            jax.ShapeDtypeStruct((n_tokens, NUM_EXPERTS), jnp.float32),
            jax.ShapeDtypeStruct((n_tokens, TOP_K), jnp.int32),
        ],
    )(x, Wg.T, Wn.T, noise)
    return gates, experts
